# baseline (speedup 1.0000x reference)
import functools

import jax
import jax.numpy as jnp
from jax import lax
from jax.experimental import pallas as pl
from jax.experimental.pallas import tpu as pltpu
from jax.experimental.pallas import tpu_sc as plsc

N = 10000
F = 128
E = 320000
NC = 2
NS = 16
NW = NC * NS
EPW = E // NW
CH = 128
NCH = 80
PAD = NCH * CH - EPW
NPH = 2
PCH = NCH // NPH
UNR = 20
NP = 10240
STRIPE = NP // NS
CROWS = NP // F


def _seg_mesh():
  return plsc.VectorSubcoreMesh(
      core_axis_name="c", subcore_axis_name="s",
      num_cores=NC, num_subcores=NS)


@functools.lru_cache(None)
def _make_seg3():
  outs = [jax.ShapeDtypeStruct((NC, NP, F), jnp.float32)] * 3
  scratch = [
      pltpu.VMEM((PCH, CH), jnp.int32),
      pltpu.VMEM((PCH, CH), jnp.int32),
      pltpu.VMEM((CH, F), jnp.float32),
      pltpu.VMEM((CH, F), jnp.float32),
      pltpu.VMEM_SHARED((NP, F), jnp.float32),
      pltpu.SemaphoreType.DMA,
      pltpu.SemaphoreType.DMA,
      pltpu.SemaphoreType.DMA,
      pltpu.SemaphoreType.DMA,
  ]

  def body(t0, t1, t2, s0, d0, s1, d1, s2, d2, z128,
           o0, o1, o2, src_v, dst_v, rows0, rows1, acc, g0, g1, x0, x1):
    cid = lax.axis_index("c")
    sid = lax.axis_index("s")
    wid = sid * NC + cid
    tabs = (t0, t1, t2)
    srcs = (s0, s1, s2)
    dsts = (d0, d1, d2)
    souts = (o0, o1, o2)
    rows = (rows0, rows1)
    gsem = (g0, g1)
    ssem = (x0, x1)
    for r in range(3):
      pltpu.sync_copy(z128, acc.at[pl.ds(sid * STRIPE, STRIPE)])
      plsc.subcore_barrier()

      tab = tabs[r]
      for p in range(NPH):
        pltpu.sync_copy(srcs[r].at[wid, pl.ds(p * PCH, PCH)], src_v)
        pltpu.sync_copy(dsts[r].at[wid, pl.ds(p * PCH, PCH)], dst_v)

        @pl.loop(0, PCH // UNR)
        def _group(g):
          j0 = g * UNR
          gds = [pltpu.async_copy(tab.at[src_v.at[j0]], rows0, g0),
                 pltpu.async_copy(tab.at[src_v.at[j0 + 1]], rows1, g1)]
          sds = []
          for k in range(UNR):
            b = k & 1
            gds[k].wait()
            sds.append(pltpu.async_copy(rows[b], acc.at[dst_v.at[j0 + k]],
                                        ssem[b], add=True))
            if k + 2 < UNR:
              sds[k].wait()
              gds.append(pltpu.async_copy(
                  tab.at[src_v.at[j0 + k + 2]], rows[b], gsem[b]))
          sds[UNR - 2].wait()
          sds[UNR - 1].wait()

      plsc.subcore_barrier()
      pltpu.sync_copy(acc.at[pl.ds(sid * STRIPE, STRIPE)],
                      souts[r].at[cid, pl.ds(sid * STRIPE, STRIPE)])

  return pl.kernel(body, out_type=tuple(outs), mesh=_seg_mesh(),
                   scratch_types=scratch)


@functools.lru_cache(None)
def _make_counts_tc():
  T = 2000
  grid = (E // T,)

  def body(d0, d1, d2, o0, o1, o2):
    il = lax.broadcasted_iota(jnp.int32, (1, F), 1)
    ih = lax.broadcasted_iota(jnp.int32, (1, CROWS), 1)
    for d, o in ((d0, o0), (d1, o1), (d2, o2)):
      dd = d[...]
      ol = (jnp.bitwise_and(dd, F - 1) == il).astype(jnp.bfloat16)
      oh = (lax.shift_right_logical(dd, 7) == ih).astype(jnp.bfloat16)
      part = lax.dot_general(oh, ol, (((0,), (0,)), ((), ())),
                             preferred_element_type=jnp.float32)

      @pl.when(pl.program_id(0) == 0)
      def _():
        o[...] = jnp.zeros_like(o)

      o[...] += part

  d_spec = pl.BlockSpec((T, 1), lambda i: (i, 0))
  o_spec = pl.BlockSpec((CROWS, F), lambda i: (0, 0))
  return pl.pallas_call(
      body,
      grid=grid,
      in_specs=[d_spec] * 3,
      out_specs=[o_spec] * 3,
      out_shape=[jax.ShapeDtypeStruct((CROWS, F), jnp.float32)] * 3,
  )


def _make_dense(relu):
  BLK = 1000
  grid = (N // BLK,)

  def body(sr, cr, sa, ca, su, cu, xi, xu,
           wlr, blr, wrr, wla, bla, wra, wlu, blu, wru,
           item_o, user_o):
    def agg(s_ref, c_ref):
      s = s_ref[...]
      cnt = jnp.maximum(c_ref[...], 1.0)
      return (s[0] + s[1]) / cnt

    dot = functools.partial(jnp.dot, preferred_element_type=jnp.float32)
    it = (dot(agg(sr, cr), wlr[...]) + blr[...] + dot(xi[...], wrr[...])
          + dot(agg(sa, ca), wla[...]) + bla[...] + dot(xi[...], wra[...]))
    us = dot(agg(su, cu), wlu[...]) + blu[...] + dot(xu[...], wru[...])
    if relu:
      it = jnp.maximum(it, 0.0)
      us = jnp.maximum(us, 0.0)
    item_o[...] = it
    user_o[...] = us

  s_spec = pl.BlockSpec((NC, BLK, F), lambda i: (0, i, 0))
  c_spec = pl.BlockSpec((BLK, 1), lambda i: (i, 0))
  x_spec = pl.BlockSpec((BLK, F), lambda i: (i, 0))
  w_spec = pl.BlockSpec((F, F), lambda i: (0, 0))
  b_spec = pl.BlockSpec((1, F), lambda i: (0, 0))
  return pl.pallas_call(
      body,
      grid=grid,
      in_specs=[s_spec, c_spec, s_spec, c_spec, s_spec, c_spec,
                x_spec, x_spec,
                w_spec, b_spec, w_spec, w_spec, b_spec, w_spec,
                w_spec, b_spec, w_spec],
      out_specs=[x_spec, x_spec],
      out_shape=[jax.ShapeDtypeStruct((N, F), jnp.float32)] * 2,
  )


_dense_relu = _make_dense(True)
_dense_out = _make_dense(False)


def kernel(x_user, x_item, edge_index_reviews, edge_index_rev_reviews,
           edge_index_also_bought,
           l1_rev_Wl, l1_rev_bl, l1_rev_Wr, l1_rr_Wl, l1_rr_bl, l1_rr_Wr,
           l1_ab_Wl, l1_ab_bl, l1_ab_Wr,
           l2_rev_Wl, l2_rev_bl, l2_rev_Wr, l2_rr_Wl, l2_rr_bl, l2_rr_Wr,
           l2_ab_Wl, l2_ab_bl, l2_ab_Wr):
  def split(ei):
    ei = ei.astype(jnp.int32)
    s = ei[0].reshape(NW, EPW)
    d = ei[1].reshape(NW, EPW)
    pad_s = jnp.broadcast_to(jnp.arange(PAD, dtype=jnp.int32), (NW, PAD))
    pad_d = N + pad_s
    s = jnp.concatenate([s, pad_s], axis=1)
    d = jnp.concatenate([d, pad_d], axis=1)
    return s.reshape(NW, NCH, CH), d.reshape(NW, NCH, CH)

  src_rev, dst_rev = split(edge_index_reviews)
  src_rr, dst_rr = split(edge_index_rev_reviews)
  src_ab, dst_ab = split(edge_index_also_bought)

  z128 = jnp.zeros((STRIPE, F), jnp.float32)

  def dcol(ei):
    return ei[1].astype(jnp.int32).reshape(E, 1)

  c_rev, c_ab, c_rr = _make_counts_tc()(
      dcol(edge_index_reviews), dcol(edge_index_also_bought),
      dcol(edge_index_rev_reviews))

  def cfmt(c):
    return c.reshape(NP, 1)

  c_rev = cfmt(c_rev)
  c_ab = cfmt(c_ab)
  c_rr = cfmt(c_rr)

  s_rev, s_ab, s_rr = _make_seg3()(
      x_user, x_item, x_item,
      src_rev, dst_rev, src_ab, dst_ab, src_rr, dst_rr, z128)

  def p(w):
    return w.T

  def b(v):
    return v.reshape(1, F)

  item1, user1 = _dense_relu(
      s_rev, c_rev, s_ab, c_ab, s_rr, c_rr, x_item, x_user,
      p(l1_rev_Wl), b(l1_rev_bl), p(l1_rev_Wr),
      p(l1_ab_Wl), b(l1_ab_bl), p(l1_ab_Wr),
      p(l1_rr_Wl), b(l1_rr_bl), p(l1_rr_Wr))

  s2_rev, s2_ab, s2_rr = _make_seg3()(
      user1, item1, item1,
      src_rev, dst_rev, src_ab, dst_ab, src_rr, dst_rr, z128)

  item2, user2 = _dense_out(
      s2_rev, c_rev, s2_ab, c_ab, s2_rr, c_rr, item1, user1,
      p(l2_rev_Wl), b(l2_rev_bl), p(l2_rev_Wr),
      p(l2_ab_Wl), b(l2_ab_bl), p(l2_ab_Wr),
      p(l2_rr_Wl), b(l2_rr_bl), p(l2_rr_Wr))

  return (user2, item2)


# speedup vs baseline: 1.0637x; 1.0637x over previous
import functools

import jax
import jax.numpy as jnp
from jax import lax
from jax.experimental import pallas as pl
from jax.experimental.pallas import tpu as pltpu
from jax.experimental.pallas import tpu_sc as plsc

N = 10000
F = 128
E = 320000
NC = 2
NS = 16
NW = NC * NS
EPW = E // NW
CH = 128
NCH = 80
PAD = NCH * CH - EPW
NPH = 2
PCH = NCH // NPH
UNR = 20
NP = 10240
STRIPE = NP // NS
CROWS = NP // F


def _seg_mesh():
  return plsc.VectorSubcoreMesh(
      core_axis_name="c", subcore_axis_name="s",
      num_cores=NC, num_subcores=NS)


@functools.lru_cache(None)
def _make_seg3():
  outs = [jax.ShapeDtypeStruct((NC, NP, F), jnp.float32)] * 3
  scratch = [
      pltpu.VMEM((PCH, CH), jnp.int32),
      pltpu.VMEM((PCH, CH), jnp.int32),
      pltpu.VMEM((CH, F), jnp.float32),
      pltpu.VMEM((CH, F), jnp.float32),
      pltpu.VMEM_SHARED((NP, F), jnp.float32),
      pltpu.SemaphoreType.DMA,
      pltpu.SemaphoreType.DMA,
      pltpu.SemaphoreType.DMA,
      pltpu.SemaphoreType.DMA,
  ]

  def body(t0, t1, t2, s0, d0, s1, d1, s2, d2, z128,
           o0, o1, o2, src_v, dst_v, rows0, rows1, acc, g0, g1, x0, x1):
    cid = lax.axis_index("c")
    sid = lax.axis_index("s")
    wid = sid * NC + cid
    tabs = (t0, t1, t2)
    srcs = (s0, s1, s2)
    dsts = (d0, d1, d2)
    souts = (o0, o1, o2)
    rows = (rows0, rows1)
    gsem = (g0, g1)
    ssem = (x0, x1)
    for r in range(3):
      pltpu.sync_copy(z128, acc.at[pl.ds(sid * STRIPE, STRIPE)])
      plsc.subcore_barrier()

      tab = tabs[r]
      for p in range(NPH):
        pltpu.sync_copy(srcs[r].at[wid, pl.ds(p * PCH, PCH)], src_v)
        pltpu.sync_copy(dsts[r].at[wid, pl.ds(p * PCH, PCH)], dst_v)

        @pl.loop(0, PCH // UNR)
        def _group(g):
          j0 = g * UNR
          gds = [pltpu.async_copy(tab.at[src_v.at[j0]], rows0, g0),
                 pltpu.async_copy(tab.at[src_v.at[j0 + 1]], rows1, g1)]
          sds = []
          for k in range(UNR):
            b = k & 1
            gds[k].wait()
            sds.append(pltpu.async_copy(rows[b], acc.at[dst_v.at[j0 + k]],
                                        ssem[b], add=True))
            if k + 2 < UNR:
              sds[k].wait()
              gds.append(pltpu.async_copy(
                  tab.at[src_v.at[j0 + k + 2]], rows[b], gsem[b]))
          sds[UNR - 2].wait()
          sds[UNR - 1].wait()

      plsc.subcore_barrier()
      pltpu.sync_copy(acc.at[pl.ds(sid * STRIPE, STRIPE)],
                      souts[r].at[cid, pl.ds(sid * STRIPE, STRIPE)])

  return pl.kernel(body, out_type=tuple(outs), mesh=_seg_mesh(),
                   scratch_types=scratch)


@functools.lru_cache(None)
def _make_counts_tc():
  T = 8000
  grid = (E // T,)

  def body(d0, d1, d2, o0, o1, o2):
    il = lax.broadcasted_iota(jnp.int32, (1, F), 1)
    ih = lax.broadcasted_iota(jnp.int32, (1, CROWS), 1)
    for d, o in ((d0, o0), (d1, o1), (d2, o2)):
      dd = d[...]
      ol = (jnp.bitwise_and(dd, F - 1) == il).astype(jnp.bfloat16)
      oh = (lax.shift_right_logical(dd, 7) == ih).astype(jnp.bfloat16)
      part = lax.dot_general(oh, ol, (((0,), (0,)), ((), ())),
                             preferred_element_type=jnp.float32)

      @pl.when(pl.program_id(0) == 0)
      def _():
        o[...] = jnp.zeros_like(o)

      o[...] += part

  d_spec = pl.BlockSpec((T, 1), lambda i: (i, 0))
  o_spec = pl.BlockSpec((CROWS, F), lambda i: (0, 0))
  return pl.pallas_call(
      body,
      grid=grid,
      in_specs=[d_spec] * 3,
      out_specs=[o_spec] * 3,
      out_shape=[jax.ShapeDtypeStruct((CROWS, F), jnp.float32)] * 3,
  )


def _make_dense(relu):
  BLK = 1000
  grid = (N // BLK,)

  def body(sr, cr, sa, ca, su, cu, xi, xu,
           wlr, blr, wrr, wla, bla, wra, wlu, blu, wru,
           item_o, user_o):
    def agg(s_ref, c_ref):
      s = s_ref[...]
      cnt = jnp.maximum(c_ref[...], 1.0)
      return (s[0] + s[1]) / cnt

    def dot(a, w_ref):
      return lax.dot_general(a, w_ref[...], (((1,), (1,)), ((), ())),
                             preferred_element_type=jnp.float32)

    it = (dot(agg(sr, cr), wlr) + blr[...] + dot(xi[...], wrr)
          + dot(agg(sa, ca), wla) + bla[...] + dot(xi[...], wra))
    us = dot(agg(su, cu), wlu) + blu[...] + dot(xu[...], wru)
    if relu:
      it = jnp.maximum(it, 0.0)
      us = jnp.maximum(us, 0.0)
    item_o[...] = it
    user_o[...] = us

  s_spec = pl.BlockSpec((NC, BLK, F), lambda i: (0, i, 0))
  c_spec = pl.BlockSpec((BLK, 1), lambda i: (i, 0))
  x_spec = pl.BlockSpec((BLK, F), lambda i: (i, 0))
  w_spec = pl.BlockSpec((F, F), lambda i: (0, 0))
  b_spec = pl.BlockSpec((1, F), lambda i: (0, 0))
  return pl.pallas_call(
      body,
      grid=grid,
      in_specs=[s_spec, c_spec, s_spec, c_spec, s_spec, c_spec,
                x_spec, x_spec,
                w_spec, b_spec, w_spec, w_spec, b_spec, w_spec,
                w_spec, b_spec, w_spec],
      out_specs=[x_spec, x_spec],
      out_shape=[jax.ShapeDtypeStruct((N, F), jnp.float32)] * 2,
  )


_dense_relu = _make_dense(True)
_dense_out = _make_dense(False)


def kernel(x_user, x_item, edge_index_reviews, edge_index_rev_reviews,
           edge_index_also_bought,
           l1_rev_Wl, l1_rev_bl, l1_rev_Wr, l1_rr_Wl, l1_rr_bl, l1_rr_Wr,
           l1_ab_Wl, l1_ab_bl, l1_ab_Wr,
           l2_rev_Wl, l2_rev_bl, l2_rev_Wr, l2_rr_Wl, l2_rr_bl, l2_rr_Wr,
           l2_ab_Wl, l2_ab_bl, l2_ab_Wr):
  def split(ei):
    ei = ei.astype(jnp.int32)
    s = ei[0].reshape(NW, EPW)
    d = ei[1].reshape(NW, EPW)
    pad_s = jnp.broadcast_to(jnp.arange(PAD, dtype=jnp.int32), (NW, PAD))
    pad_d = N + pad_s
    s = jnp.concatenate([s, pad_s], axis=1)
    d = jnp.concatenate([d, pad_d], axis=1)
    return s.reshape(NW, NCH, CH), d.reshape(NW, NCH, CH)

  src_rev, dst_rev = split(edge_index_reviews)
  src_rr, dst_rr = split(edge_index_rev_reviews)
  src_ab, dst_ab = split(edge_index_also_bought)

  z128 = jnp.zeros((STRIPE, F), jnp.float32)

  def dcol(ei):
    return ei[1].astype(jnp.int32).reshape(E, 1)

  c_rev, c_ab, c_rr = _make_counts_tc()(
      dcol(edge_index_reviews), dcol(edge_index_also_bought),
      dcol(edge_index_rev_reviews))

  def cfmt(c):
    return c.reshape(NP, 1)

  c_rev = cfmt(c_rev)
  c_ab = cfmt(c_ab)
  c_rr = cfmt(c_rr)

  s_rev, s_ab, s_rr = _make_seg3()(
      x_user, x_item, x_item,
      src_rev, dst_rev, src_ab, dst_ab, src_rr, dst_rr, z128)

  def b(v):
    return v.reshape(1, F)

  item1, user1 = _dense_relu(
      s_rev, c_rev, s_ab, c_ab, s_rr, c_rr, x_item, x_user,
      l1_rev_Wl, b(l1_rev_bl), l1_rev_Wr,
      l1_ab_Wl, b(l1_ab_bl), l1_ab_Wr,
      l1_rr_Wl, b(l1_rr_bl), l1_rr_Wr)

  s2_rev, s2_ab, s2_rr = _make_seg3()(
      user1, item1, item1,
      src_rev, dst_rev, src_ab, dst_ab, src_rr, dst_rr, z128)

  item2, user2 = _dense_out(
      s2_rev, c_rev, s2_ab, c_ab, s2_rr, c_rr, item1, user1,
      l2_rev_Wl, b(l2_rev_bl), l2_rev_Wr,
      l2_ab_Wl, b(l2_ab_bl), l2_ab_Wr,
      l2_rr_Wl, b(l2_rr_bl), l2_rr_Wr)

  return (user2, item2)
